# Initial kernel scaffold; baseline (speedup 1.0000x reference)
import jax
import jax.numpy as jnp
from jax.experimental import pallas as pl


def kernel(edge_index, edge_vals, input_feature, weight, bias):
    raise NotImplementedError("write your pallas kernel here")


# speedup vs baseline: 3.6432x; 3.6432x over previous
import functools

import jax
import jax.numpy as jnp
from jax import lax
from jax.experimental import pallas as pl
from jax.experimental.pallas import tpu as pltpu
from jax.experimental.pallas import tpu_sc as plsc

N_NODES = 10000
N_EDGES = 160000
D_HALF = 128

NUM_CORES = 2
NUM_SUBCORES = 16
E_PER_TILE = N_EDGES // NUM_SUBCORES
CHUNK = 80
N_CHUNKS = E_PER_TILE // CHUNK
N_PAD = 10240
ROWS_PER_TILE = N_PAD // NUM_SUBCORES
INIT_ROWS = 128


def _spmm_kernel(xa, xb, eidx, ev3):
    mesh = plsc.VectorSubcoreMesh(core_axis_name="c", subcore_axis_name="s")

    @functools.partial(
        pl.kernel,
        mesh=mesh,
        out_type=(
            jax.ShapeDtypeStruct((N_PAD, D_HALF), jnp.float32),
            jax.ShapeDtypeStruct((N_PAD, D_HALF), jnp.float32),
        ),
        scratch_types=[
            pltpu.VMEM((2, CHUNK), jnp.int32),
            pltpu.VMEM((E_PER_TILE,), jnp.float32),
            pltpu.VMEM((CHUNK, D_HALF), jnp.float32),
            pltpu.VMEM_SHARED((N_PAD, D_HALF), jnp.float32),
            pltpu.SemaphoreType.DMA,
        ],
    )
    def k(xa_ref, xb_ref, eidx_ref, ev_ref, ha_ref, hb_ref,
          idxc, evv, rowbuf, hi_sh, sem):
        c = lax.axis_index("c")
        s = lax.axis_index("s")

        pltpu.sync_copy(ev_ref.at[s], evv)

        zero16 = jnp.zeros((16,), jnp.float32)

        def zrow(r, carry):
            for j in range(D_HALF // 16):
                rowbuf[r, pl.ds(j * 16, 16)] = zero16
            return carry

        lax.fori_loop(0, CHUNK, zrow, 0)
        for i in range(ROWS_PER_TILE // CHUNK):
            base = s * ROWS_PER_TILE + i * CHUNK
            pltpu.sync_copy(rowbuf, hi_sh.at[pl.ds(base, CHUNK)])
        plsc.subcore_barrier()

        def accumulate(table_ref, out_ref):
            def chunk(j, carry):
                pltpu.sync_copy(eidx_ref.at[s, j], idxc)
                pltpu.async_copy(table_ref.at[idxc.at[0]], rowbuf, sem).wait()

                def edge_group(g, carry2):
                    evt = evv[pl.ds(j * CHUNK + g * 16, 16)]
                    for e16 in range(16):
                        ev16 = lax.gather(
                            evt,
                            jnp.full((16, 1), e16, jnp.int32),
                            lax.GatherDimensionNumbers(
                                offset_dims=(),
                                collapsed_slice_dims=(0,),
                                start_index_map=(0,),
                            ),
                            (1,),
                            mode=lax.GatherScatterMode.PROMISE_IN_BOUNDS,
                        )
                        e = g * 16 + e16
                        for jj in range(D_HALF // 16):
                            sl = pl.ds(jj * 16, 16)
                            rowbuf[e, sl] = rowbuf[e, sl] * ev16
                    return carry2

                lax.fori_loop(0, CHUNK // 16, edge_group, 0)

                pltpu.sync_copy(rowbuf, hi_sh.at[idxc.at[1]], add=True)
                return carry

            lax.fori_loop(0, N_CHUNKS, chunk, 0)
            plsc.subcore_barrier()

            for i in range(ROWS_PER_TILE // INIT_ROWS):
                base = s * ROWS_PER_TILE + i * INIT_ROWS
                pltpu.sync_copy(hi_sh.at[pl.ds(base, INIT_ROWS)],
                                out_ref.at[pl.ds(base, INIT_ROWS)])

        @pl.when(c == 0)
        def _():
            accumulate(xa_ref, ha_ref)

        @pl.when(c == 1)
        def _():
            accumulate(xb_ref, hb_ref)

    return k(xa, xb, eidx, ev3)


def _mm_body(ha_ref, hb_ref, wa_ref, wb_ref, b_ref, o_ref):
    acc = jnp.dot(ha_ref[...], wa_ref[...], preferred_element_type=jnp.float32)
    acc = acc + jnp.dot(hb_ref[...], wb_ref[...],
                        preferred_element_type=jnp.float32)
    o_ref[...] = acc + b_ref[...]


def _matmul(ha, hb, wa, wb, bias2):
    n, d_out = N_NODES, wa.shape[1]
    blk = 1000
    return pl.pallas_call(
        _mm_body,
        grid=(n // blk,),
        in_specs=[
            pl.BlockSpec((blk, D_HALF), lambda i: (i, 0)),
            pl.BlockSpec((blk, D_HALF), lambda i: (i, 0)),
            pl.BlockSpec((D_HALF, d_out), lambda i: (0, 0)),
            pl.BlockSpec((D_HALF, d_out), lambda i: (0, 0)),
            pl.BlockSpec((1, d_out), lambda i: (0, 0)),
        ],
        out_specs=pl.BlockSpec((blk, d_out), lambda i: (i, 0)),
        out_shape=jax.ShapeDtypeStruct((n, d_out), jnp.float32),
    )(ha, hb, wa, wb, bias2)


def kernel(edge_index, edge_vals, input_feature, weight, bias):
    ei = edge_index.astype(jnp.int32)
    row3 = ei[0].reshape(NUM_SUBCORES, N_CHUNKS, CHUNK)
    col3 = ei[1].reshape(NUM_SUBCORES, N_CHUNKS, CHUNK)
    eidx = jnp.stack([col3, row3], axis=2)
    ev3 = edge_vals.astype(jnp.float32).reshape(NUM_SUBCORES, E_PER_TILE)
    xa = input_feature[:, :D_HALF]
    xb = input_feature[:, D_HALF:]
    ha, hb = _spmm_kernel(xa, xb, eidx, ev3)
    return _matmul(ha, hb, weight[:D_HALF], weight[D_HALF:],
                   bias.reshape(1, -1))
